# baseline (speedup 1.0000x reference)
import jax
import jax.numpy as jnp
import numpy as np
from jax.experimental import pallas as pl
from jax.experimental.pallas import tpu as pltpu

N = 4096
D_IN = 256
H = 128
C = 32
CE = C + 2
K = 4
LAM_HAT = 0.9
A_SCAD = 3.7
EPS = 1e-8

BT = 512
NT = N // BT
NPAIRS = NT * (NT + 1) // 2
BP = 256


def _augment(xn):
    sq = jnp.sum(xn * xn, axis=1, keepdims=True)
    ones = jnp.ones_like(sq)
    xna = jnp.concatenate([xn * -2.0, sq, ones], axis=1)
    xnb = jnp.concatenate([xn, ones, sq], axis=1)
    return xna, xnb


def _prep_kernel(A_ref, F_ref, W1_ref, b1_ref, W2_ref, b2_ref,
                 F0_ref, Dd_ref, dinv_ref, XnA_ref, XnB_ref):
    a = A_ref[...]
    dd = jnp.sum(a, axis=1, keepdims=True) + 1.0
    Dd_ref[...] = dd
    dinv = jax.lax.rsqrt(dd)
    dinv_ref[...] = dinv
    h = jnp.maximum(
        jnp.dot(F_ref[...], W1_ref[...], preferred_element_type=jnp.float32)
        + b1_ref[...], 0.0)
    f0 = (jnp.dot(h, W2_ref[...], preferred_element_type=jnp.float32)
          + b2_ref[...])
    F0_ref[...] = f0
    xna, xnb = _augment(f0 * dinv)
    XnA_ref[...] = xna
    XnB_ref[...] = xnb


def _iter_kernel(ti_ref, tj_ref, lam_ref, A1_ref, At1_ref, XnA_ref, XnB_ref,
                 dinv_ref, Dd_ref, F0_ref,
                 out_ref, oXnA_ref, oXnB_ref, P_acc):
    p = pl.program_id(0)
    ti = ti_ref[p]
    tj = tj_ref[p]
    lam_k = lam_ref[0]
    lam = 1.0 / LAM_HAT - 1.0
    alam = A_SCAD * lam_k
    inv_c = 1.0 / (2.0 * (A_SCAD - 1.0) * lam_k)

    @pl.when(p == 0)
    def _():
        P_acc[...] = jnp.zeros_like(P_acc)

    @pl.when(p < NPAIRS)
    def _():
        ai = XnA_ref[pl.ds(ti * BT, BT), :]
        bj = XnB_ref[pl.ds(tj * BT, BT), :]
        zpre = jax.lax.dot_general(ai, bj, (((1,), (1,)), ((), ())),
                                   preferred_element_type=jnp.float32)
        z = jnp.maximum(zpre, 0.0)
        r = jax.lax.rsqrt(jnp.maximum(z, EPS * EPS))
        y = z * r
        t = jnp.maximum(jnp.minimum(alam * inv_c - y * inv_c, 0.5), 0.0)
        w = t * r
        row = jax.lax.broadcasted_iota(jnp.int32, (BT, BT), 0)
        col = jax.lax.broadcasted_iota(jnp.int32, (BT, BT), 1)
        w = jnp.where(jnp.logical_and(ti == tj, row == col), 0.0, w)

        wa1 = w * A1_ref[...]
        P_acc[pl.ds(ti * BT, BT), :] += jax.lax.dot_general(
            wa1, XnB_ref[pl.ds(tj * BT, BT), :C + 1], (((1,), (0,)), ((), ())),
            preferred_element_type=jnp.float32)

        @pl.when(ti != tj)
        def _():
            u = w * At1_ref[...]
            P_acc[pl.ds(tj * BT, BT), :] += jax.lax.dot_general(
                u, XnB_ref[pl.ds(ti * BT, BT), :C + 1],
                (((0,), (0,)), ((), ())),
                preferred_element_type=jnp.float32)

    @pl.when(p == NPAIRS)
    def _():
        q = P_acc[:, C:C + 1] / Dd_ref[...] + lam
        dinv = dinv_ref[...]
        fc = (dinv * P_acc[:, :C] + lam * F0_ref[...]) / q
        out_ref[...] = fc
        xna, xnb = _augment(fc * dinv)
        oXnA_ref[...] = xna
        oXnB_ref[...] = xnb


def _prep_call(A, F, W1, b1, W2, b2):
    return pl.pallas_call(
        _prep_kernel,
        grid=(N // BP,),
        in_specs=[
            pl.BlockSpec((BP, N), lambda i: (i, 0)),
            pl.BlockSpec((BP, D_IN), lambda i: (i, 0)),
            pl.BlockSpec((D_IN, H), lambda i: (0, 0)),
            pl.BlockSpec((1, H), lambda i: (0, 0)),
            pl.BlockSpec((H, C), lambda i: (0, 0)),
            pl.BlockSpec((1, C), lambda i: (0, 0)),
        ],
        out_specs=[
            pl.BlockSpec((BP, C), lambda i: (i, 0)),
            pl.BlockSpec((BP, 1), lambda i: (i, 0)),
            pl.BlockSpec((BP, 1), lambda i: (i, 0)),
            pl.BlockSpec((BP, CE), lambda i: (i, 0)),
            pl.BlockSpec((BP, CE), lambda i: (i, 0)),
        ],
        out_shape=[
            jax.ShapeDtypeStruct((N, C), jnp.float32),
            jax.ShapeDtypeStruct((N, 1), jnp.float32),
            jax.ShapeDtypeStruct((N, 1), jnp.float32),
            jax.ShapeDtypeStruct((N, CE), jnp.float32),
            jax.ShapeDtypeStruct((N, CE), jnp.float32),
        ],
        compiler_params=pltpu.CompilerParams(
            dimension_semantics=("arbitrary",)),
    )(A, F, W1, b1, W2, b2)


_TI_LIST = []
_TJ_LIST = []
for _a in range(NT):
    for _b in range(_a, NT):
        _TI_LIST.append(_a)
        _TJ_LIST.append(_b)
_TI_LIST.append(0)
_TJ_LIST.append(0)
_TI_ARR = np.asarray(_TI_LIST, np.int32)
_TJ_ARR = np.asarray(_TJ_LIST, np.int32)


def _iter_call(lam_k, A, At, XnA, XnB, dinv, Dd, F0):
    grid_spec = pltpu.PrefetchScalarGridSpec(
        num_scalar_prefetch=3,
        grid=(NPAIRS + 1,),
        in_specs=[
            pl.BlockSpec((BT, BT), lambda p, ti, tj, lam: (ti[p], tj[p])),
            pl.BlockSpec((BT, BT), lambda p, ti, tj, lam: (ti[p], tj[p])),
            pl.BlockSpec((N, CE), lambda p, ti, tj, lam: (0, 0)),
            pl.BlockSpec((N, CE), lambda p, ti, tj, lam: (0, 0)),
            pl.BlockSpec((N, 1), lambda p, ti, tj, lam: (0, 0)),
            pl.BlockSpec((N, 1), lambda p, ti, tj, lam: (0, 0)),
            pl.BlockSpec((N, C), lambda p, ti, tj, lam: (0, 0)),
        ],
        out_specs=[
            pl.BlockSpec((N, C), lambda p, ti, tj, lam: (0, 0)),
            pl.BlockSpec((N, CE), lambda p, ti, tj, lam: (0, 0)),
            pl.BlockSpec((N, CE), lambda p, ti, tj, lam: (0, 0)),
        ],
        scratch_shapes=[
            pltpu.VMEM((N, C + 1), jnp.float32),
        ],
    )
    return pl.pallas_call(
        _iter_kernel,
        grid_spec=grid_spec,
        out_shape=[
            jax.ShapeDtypeStruct((N, C), jnp.float32),
            jax.ShapeDtypeStruct((N, CE), jnp.float32),
            jax.ShapeDtypeStruct((N, CE), jnp.float32),
        ],
        compiler_params=pltpu.CompilerParams(
            dimension_semantics=("arbitrary",)),
    )(jnp.asarray(_TI_ARR), jnp.asarray(_TJ_ARR), lam_k,
      A, At, XnA, XnB, dinv, Dd, F0)


def kernel(A, F, W1, b1, W2, b2, log_lams):
    F0, Dd, dinv, XnA, XnB = _prep_call(
        A, F, W1, b1.reshape(1, H), W2, b2.reshape(1, C))
    At = A.T
    lams = jnp.exp(log_lams)
    Fc = F0
    for k in range(K):
        Fc, XnA, XnB = _iter_call(
            lams[k].reshape(1), A, At, XnA, XnB, dinv, Dd, F0)
    return Fc


# speedup vs baseline: 1.0099x; 1.0099x over previous
import jax
import jax.numpy as jnp
import numpy as np
from jax.experimental import pallas as pl
from jax.experimental.pallas import tpu as pltpu

N = 4096
D_IN = 256
H = 128
C = 32
CE = C + 2
CP = C + 1
K = 4
LAM_HAT = 0.9
A_SCAD = 3.7
EPS = 1e-8

BT = 512
NT = N // BT
NPAIRS = NT * (NT + 1) // 2
BP = 256


def _augment(xn):
    sq = jnp.sum(xn * xn, axis=1, keepdims=True)
    ones = jnp.ones_like(sq)
    xna = jnp.concatenate([xn * -2.0, sq, ones], axis=1)
    xnb = jnp.concatenate([xn, ones, sq], axis=1)
    return xna, xnb


def _prep_kernel(A_ref, F_ref, W1_ref, b1_ref, W2_ref, b2_ref,
                 F0_ref, Dd_ref, dinv_ref, XnA_ref, XnB_ref, XnBT_ref):
    a = A_ref[...]
    dd = jnp.sum(a, axis=1, keepdims=True) + 1.0
    Dd_ref[...] = dd
    dinv = jax.lax.rsqrt(dd)
    dinv_ref[...] = dinv
    h = jnp.maximum(
        jnp.dot(F_ref[...], W1_ref[...], preferred_element_type=jnp.float32)
        + b1_ref[...], 0.0)
    f0 = (jnp.dot(h, W2_ref[...], preferred_element_type=jnp.float32)
          + b2_ref[...])
    F0_ref[...] = f0
    xna, xnb = _augment(f0 * dinv)
    XnA_ref[...] = xna
    XnB_ref[...] = xnb
    XnBT_ref[...] = xnb.T


def _iter_kernel(ti_ref, tj_ref, lam_ref, A1_ref, At1_ref, XnA_ref, XnB_ref,
                 XnBT_ref, dinv_ref, Dd_ref, F0_ref,
                 out_ref, oXnA_ref, oXnB_ref, oXnBT_ref, P_acc, P2T_acc):
    p = pl.program_id(0)
    ti = ti_ref[p]
    tj = tj_ref[p]
    lam_k = lam_ref[0]
    lam = 1.0 / LAM_HAT - 1.0
    alam = A_SCAD * lam_k
    inv_c = 1.0 / (2.0 * (A_SCAD - 1.0) * lam_k)

    @pl.when(p == 0)
    def _():
        P_acc[...] = jnp.zeros_like(P_acc)
        P2T_acc[...] = jnp.zeros_like(P2T_acc)

    @pl.when(p < NPAIRS)
    def _():
        ai = XnA_ref[pl.ds(ti * BT, BT), :]
        bj = XnB_ref[pl.ds(tj * BT, BT), :]
        zpre = jax.lax.dot_general(ai, bj, (((1,), (1,)), ((), ())),
                                   preferred_element_type=jnp.float32)
        z = jnp.maximum(zpre, 0.0)
        r = jax.lax.rsqrt(jnp.maximum(z, EPS * EPS))
        y = z * r
        t = jnp.maximum(jnp.minimum(alam * inv_c - y * inv_c, 0.5), 0.0)
        w = t * r

        @pl.when(ti == tj)
        def _():
            row = jax.lax.broadcasted_iota(jnp.int32, (BT, BT), 0)
            col = jax.lax.broadcasted_iota(jnp.int32, (BT, BT), 1)
            wd = jnp.where(row == col, 0.0, w)
            wa1 = wd * A1_ref[...]
            P_acc[pl.ds(ti * BT, BT), :] += jax.lax.dot_general(
                wa1, XnB_ref[pl.ds(tj * BT, BT), :CP],
                (((1,), (0,)), ((), ())),
                preferred_element_type=jnp.float32)

        @pl.when(ti != tj)
        def _():
            wa1 = w * A1_ref[...]
            P_acc[pl.ds(ti * BT, BT), :] += jax.lax.dot_general(
                wa1, XnB_ref[pl.ds(tj * BT, BT), :CP],
                (((1,), (0,)), ((), ())),
                preferred_element_type=jnp.float32)
            u = w * At1_ref[...]
            P2T_acc[:, pl.ds(tj * BT, BT)] += jax.lax.dot_general(
                XnBT_ref[:CP, pl.ds(ti * BT, BT)], u,
                (((1,), (0,)), ((), ())),
                preferred_element_type=jnp.float32)

    @pl.when(p == NPAIRS)
    def _():
        ptot = P_acc[...] + P2T_acc[...].T
        q = ptot[:, C:C + 1] / Dd_ref[...] + lam
        dinv = dinv_ref[...]
        fc = (dinv * ptot[:, :C] + lam * F0_ref[...]) / q
        out_ref[...] = fc
        xna, xnb = _augment(fc * dinv)
        oXnA_ref[...] = xna
        oXnB_ref[...] = xnb
        oXnBT_ref[...] = xnb.T


def _prep_call(A, F, W1, b1, W2, b2):
    return pl.pallas_call(
        _prep_kernel,
        grid=(N // BP,),
        in_specs=[
            pl.BlockSpec((BP, N), lambda i: (i, 0)),
            pl.BlockSpec((BP, D_IN), lambda i: (i, 0)),
            pl.BlockSpec((D_IN, H), lambda i: (0, 0)),
            pl.BlockSpec((1, H), lambda i: (0, 0)),
            pl.BlockSpec((H, C), lambda i: (0, 0)),
            pl.BlockSpec((1, C), lambda i: (0, 0)),
        ],
        out_specs=[
            pl.BlockSpec((BP, C), lambda i: (i, 0)),
            pl.BlockSpec((BP, 1), lambda i: (i, 0)),
            pl.BlockSpec((BP, 1), lambda i: (i, 0)),
            pl.BlockSpec((BP, CE), lambda i: (i, 0)),
            pl.BlockSpec((BP, CE), lambda i: (i, 0)),
            pl.BlockSpec((CE, BP), lambda i: (0, i)),
        ],
        out_shape=[
            jax.ShapeDtypeStruct((N, C), jnp.float32),
            jax.ShapeDtypeStruct((N, 1), jnp.float32),
            jax.ShapeDtypeStruct((N, 1), jnp.float32),
            jax.ShapeDtypeStruct((N, CE), jnp.float32),
            jax.ShapeDtypeStruct((N, CE), jnp.float32),
            jax.ShapeDtypeStruct((CE, N), jnp.float32),
        ],
        compiler_params=pltpu.CompilerParams(
            dimension_semantics=("arbitrary",)),
    )(A, F, W1, b1, W2, b2)


_TI_LIST = []
_TJ_LIST = []
for _a in range(NT):
    for _b in range(_a, NT):
        _TI_LIST.append(_a)
        _TJ_LIST.append(_b)
_TI_LIST.append(0)
_TJ_LIST.append(0)
_TI_ARR = np.asarray(_TI_LIST, np.int32)
_TJ_ARR = np.asarray(_TJ_LIST, np.int32)


def _iter_call(lam_k, A, At, XnA, XnB, XnBT, dinv, Dd, F0):
    grid_spec = pltpu.PrefetchScalarGridSpec(
        num_scalar_prefetch=3,
        grid=(NPAIRS + 1,),
        in_specs=[
            pl.BlockSpec((BT, BT), lambda p, ti, tj, lam: (ti[p], tj[p])),
            pl.BlockSpec((BT, BT), lambda p, ti, tj, lam: (ti[p], tj[p])),
            pl.BlockSpec((N, CE), lambda p, ti, tj, lam: (0, 0)),
            pl.BlockSpec((N, CE), lambda p, ti, tj, lam: (0, 0)),
            pl.BlockSpec((CE, N), lambda p, ti, tj, lam: (0, 0)),
            pl.BlockSpec((N, 1), lambda p, ti, tj, lam: (0, 0)),
            pl.BlockSpec((N, 1), lambda p, ti, tj, lam: (0, 0)),
            pl.BlockSpec((N, C), lambda p, ti, tj, lam: (0, 0)),
        ],
        out_specs=[
            pl.BlockSpec((N, C), lambda p, ti, tj, lam: (0, 0)),
            pl.BlockSpec((N, CE), lambda p, ti, tj, lam: (0, 0)),
            pl.BlockSpec((N, CE), lambda p, ti, tj, lam: (0, 0)),
            pl.BlockSpec((CE, N), lambda p, ti, tj, lam: (0, 0)),
        ],
        scratch_shapes=[
            pltpu.VMEM((N, CP), jnp.float32),
            pltpu.VMEM((CP, N), jnp.float32),
        ],
    )
    return pl.pallas_call(
        _iter_kernel,
        grid_spec=grid_spec,
        out_shape=[
            jax.ShapeDtypeStruct((N, C), jnp.float32),
            jax.ShapeDtypeStruct((N, CE), jnp.float32),
            jax.ShapeDtypeStruct((N, CE), jnp.float32),
            jax.ShapeDtypeStruct((CE, N), jnp.float32),
        ],
        compiler_params=pltpu.CompilerParams(
            dimension_semantics=("arbitrary",)),
    )(jnp.asarray(_TI_ARR), jnp.asarray(_TJ_ARR), lam_k,
      A, At, XnA, XnB, XnBT, dinv, Dd, F0)


def kernel(A, F, W1, b1, W2, b2, log_lams):
    F0, Dd, dinv, XnA, XnB, XnBT = _prep_call(
        A, F, W1, b1.reshape(1, H), W2, b2.reshape(1, C))
    At = A.T
    lams = jnp.exp(log_lams)
    Fc = F0
    for k in range(K):
        Fc, XnA, XnB, XnBT = _iter_call(
            lams[k].reshape(1), A, At, XnA, XnB, XnBT, dinv, Dd, F0)
    return Fc
